# Initial kernel scaffold; baseline (speedup 1.0000x reference)
import jax
import jax.numpy as jnp
from jax.experimental import pallas as pl


def kernel(training_lut, X):
    raise NotImplementedError("write your pallas kernel here")


# speedup vs baseline: 7.6504x; 7.6504x over previous
import jax
import jax.numpy as jnp
from jax.experimental import pallas as pl
from jax.experimental.pallas import tpu as pltpu

_L = 16
_V = 2
_K = 2
_T = 256
_R = 4096
_D = 16
_S = 65536
_TV = _T // _V


def _dp_kernel(x_ref, lut0_ref, lut1_ref, cost_ref, fs_ref):
    lut0 = lut0_ref[...]
    lut1 = lut1_ref[...]

    def state_err(t):
        x0 = x_ref[0, 0, 2 * t]
        x1 = x_ref[0, 0, 2 * t + 1]
        d0 = lut0 - x0
        d1 = lut1 - x1
        return d0 * d0 + d1 * d1

    r_lin = jax.lax.broadcasted_iota(jnp.int32, (32, 128), 0) * 128 + \
        jax.lax.broadcasted_iota(jnp.int32, (32, 128), 1)

    fs_ref[0, 0] = jnp.zeros((32, 128), jnp.int32)
    cost0 = state_err(0)

    def step(t, cost):
        c4 = cost.reshape(16, 32, 128)
        bv = jnp.min(c4, axis=0)
        bi = jnp.argmin(c4, axis=0).astype(jnp.int32)
        fs_ref[0, t] = r_lin + (bi << 12)
        err = state_err(t)
        pieces = [jnp.repeat(bv[:, 8 * w:8 * w + 8], 16, axis=1)
                  for w in range(16)]
        add = jnp.stack(pieces, axis=1).reshape(512, 128)
        return err + add

    cost_fin = jax.lax.fori_loop(1, _TV, step, cost0)
    cost_ref[0] = cost_fin


def kernel(training_lut, X):
    B = X.shape[0]
    lut0 = training_lut[:, 0].reshape(512, 128)
    lut1 = training_lut[:, 1].reshape(512, 128)
    X3 = X.reshape(B, 1, _T)

    cost, fs = pl.pallas_call(
        _dp_kernel,
        grid=(B,),
        in_specs=[
            pl.BlockSpec((1, 1, _T), lambda b: (b, 0, 0),
                         memory_space=pltpu.SMEM),
            pl.BlockSpec((512, 128), lambda b: (0, 0)),
            pl.BlockSpec((512, 128), lambda b: (0, 0)),
        ],
        out_specs=[
            pl.BlockSpec((1, 512, 128), lambda b: (b, 0, 0)),
            pl.BlockSpec((1, _TV, 32, 128), lambda b: (b, 0, 0, 0)),
        ],
        out_shape=[
            jax.ShapeDtypeStruct((B, 512, 128), jnp.float32),
            jax.ShapeDtypeStruct((B, _TV, 32, 128), jnp.int32),
        ],
        compiler_params=pltpu.CompilerParams(
            dimension_semantics=("arbitrary",),
        ),
    )(X3, lut0, lut1)

    cost = cost.reshape(B, _S)
    from_state = fs.reshape(B, _TV, _R).transpose(1, 0, 2)
    return cost, from_state


# speedup vs baseline: 129.8163x; 16.9687x over previous
import jax
import jax.numpy as jnp
from jax.experimental import pallas as pl
from jax.experimental.pallas import tpu as pltpu

_L = 16
_V = 2
_K = 2
_T = 256
_R = 4096
_D = 16
_S = 65536
_TV = _T // _V


def _dp_kernel(x_ref, lut0_ref, lut1_ref, q_ref, cost_ref, fs_ref):
    lut0 = lut0_ref[...]
    lut1 = lut1_ref[...]
    q = q_ref[...]

    def state_err(t):
        x0 = x_ref[0, 0, 2 * t]
        x1 = x_ref[0, 0, 2 * t + 1]
        d0 = lut0 - x0
        d1 = lut1 - x1
        return d0 * d0 + d1 * d1

    r_lin = jax.lax.broadcasted_iota(jnp.int32, (32, 128), 0) * 128 + \
        jax.lax.broadcasted_iota(jnp.int32, (32, 128), 1)

    fs_ref[0, 0] = jnp.zeros((32, 128), jnp.int32)
    cost0 = state_err(0)

    def step(t, cost):
        c4 = cost.reshape(16, 32, 128)
        bv = jnp.min(c4, axis=0)
        bi = jnp.argmin(c4, axis=0).astype(jnp.int32)
        fs_ref[0, t] = r_lin + (bi << 12)
        err = state_err(t)
        add = jnp.dot(bv, q,
                      preferred_element_type=jnp.float32)
        add = add.reshape(32, 16, 128).reshape(512, 128)
        return err + add

    cost_fin = jax.lax.fori_loop(1, _TV, step, cost0)
    cost_ref[0] = cost_fin


def kernel(training_lut, X):
    B = X.shape[0]
    lut0 = training_lut[:, 0].reshape(512, 128)
    lut1 = training_lut[:, 1].reshape(512, 128)
    X3 = X.reshape(B, 1, _T)

    j = jnp.arange(128, dtype=jnp.int32)[:, None]
    wml = jnp.arange(2048, dtype=jnp.int32)[None, :]
    q = (j == 8 * (wml // 128) + (wml % 128) // 16).astype(jnp.float32)

    cost, fs = pl.pallas_call(
        _dp_kernel,
        grid=(B,),
        in_specs=[
            pl.BlockSpec((1, 1, _T), lambda b: (b, 0, 0),
                         memory_space=pltpu.SMEM),
            pl.BlockSpec((512, 128), lambda b: (0, 0)),
            pl.BlockSpec((512, 128), lambda b: (0, 0)),
            pl.BlockSpec((128, 2048), lambda b: (0, 0)),
        ],
        out_specs=[
            pl.BlockSpec((1, 512, 128), lambda b: (b, 0, 0)),
            pl.BlockSpec((1, _TV, 32, 128), lambda b: (b, 0, 0, 0)),
        ],
        out_shape=[
            jax.ShapeDtypeStruct((B, 512, 128), jnp.float32),
            jax.ShapeDtypeStruct((B, _TV, 32, 128), jnp.int32),
        ],
        compiler_params=pltpu.CompilerParams(
            dimension_semantics=("arbitrary",),
        ),
    )(X3, lut0, lut1, q)

    cost = cost.reshape(B, _S)
    from_state = fs.reshape(B, _TV, _R).transpose(1, 0, 2)
    return cost, from_state
